# baseline (speedup 1.0000x reference)
import functools
import math

import jax
import jax.numpy as jnp
from jax.experimental import pallas as pl
from jax.experimental.pallas import tpu as pltpu

N = 4096
D = 128
BR = 512
NB = N // BR

_CJ = (((1,), (1,)), ((), ()))
_C0 = (((0,), (0,)), ((), ()))


def _gcn_kernel(adj_ref, x_ref, w1_ref, b1_ref, w2_ref, b2_ref, w3_ref,
                b3_ref, d1w_ref, d1b_ref, d2w_ref, d2b_ref, d3w_ref,
                d3b_ref, out_ref, abuf0_ref, abuf1_ref, adj16_ref, z1t_ref,
                z2t_ref, q_ref, cs_ref, sem_ref):
    p = pl.program_id(0)

    def strip_copies(strip, buf_ref, sidx):
        return [pltpu.make_async_copy(
                    adj_ref.at[strip, :, cb, :],
                    buf_ref.at[:, cb * 128:(cb + 1) * 128],
                    sem_ref.at[sidx])
                for cb in range(32)]

    def start_strip(strip, buf_ref, sidx):
        for c in strip_copies(strip, buf_ref, sidx):
            c.start()

    def wait_strip(strip, buf_ref, sidx):
        for c in strip_copies(strip, buf_ref, sidx):
            c.wait()

    @pl.when(p == 0)
    def _init():
        start_strip(0, abuf0_ref, 0)
        z1t = jax.lax.dot_general(w1_ref[...], x_ref[...],
                                  (((0,), (1,)), ((), ())),
                                  preferred_element_type=jnp.float32)
        z1t_ref[...] = z1t.astype(jnp.bfloat16)
        z2t_ref[...] = jnp.zeros_like(z2t_ref)
        cs_ref[...] = jnp.zeros_like(cs_ref)

    def _step_body(cur_ref, cur_sidx, nxt_ref, nxt_sidx):
        @pl.when(p < NB - 1)
        def _prefetch():
            start_strip(p + 1, nxt_ref, nxt_sidx)

        wait_strip(p, cur_ref, cur_sidx)
        a2d = cur_ref[...]
        ps = jnp.sum(a2d.reshape(BR // 8, 8, N), axis=0)
        cs_ref[...] += jnp.sum(ps, axis=0, keepdims=True)
        adj16_ref[pl.ds(p * BR, BR), :] = a2d.astype(jnp.bfloat16)
        a16 = adj16_ref[pl.ds(p * BR, BR), :]
        h1 = jax.lax.dot_general(z1t_ref[...], a16, _CJ,
                                 preferred_element_type=jnp.float32)
        h1 = jnp.maximum(h1 + b1_ref[...], 0.0)
        z2c = jax.lax.dot_general(w2_ref[...], h1, _C0,
                                  preferred_element_type=jnp.float32)
        z2t_ref[:, pl.ds(p * BR, BR)] = z2c.astype(jnp.bfloat16)
        qp = jax.lax.dot_general(z2t_ref[...], a16, _CJ,
                                 preferred_element_type=jnp.float32)
        q_ref[:, pl.ds(p * BR, BR)] = qp
        z2cb = z2t_ref[:, pl.ds(p * BR, BR)]
        for i in range(NB - 1):
            @pl.when(i < p)
            def _backfill(i=i):
                ai = adj16_ref[i * BR:(i + 1) * BR, pl.ds(p * BR, BR)]
                dq = jax.lax.dot_general(z2cb, ai, _CJ,
                                         preferred_element_type=jnp.float32)
                q_ref[:, i * BR:(i + 1) * BR] += dq

    even = jax.lax.rem(p, 2) == 0

    @pl.when(jnp.logical_and(p < NB, even))
    def _step_even():
        _step_body(abuf0_ref, 0, abuf1_ref, 1)

    @pl.when(jnp.logical_and(p < NB, jnp.logical_not(even)))
    def _step_odd():
        _step_body(abuf1_ref, 1, abuf0_ref, 0)

    @pl.when(p == NB)
    def _tail():
        h2 = jnp.maximum(q_ref[...] + b2_ref[...], 0.0)
        z3 = jax.lax.dot_general(w3_ref[...], h2, _C0,
                                 preferred_element_type=jnp.float32)
        z3 = z3.astype(jnp.bfloat16).astype(jnp.float32)
        w = z3 * cs_ref[...]
        m = jnp.sum(w, axis=1, keepdims=True) * (1.0 / N)
        m = m + b3_ref[...]
        t = jax.lax.dot_general(m, d1w_ref[...], _C0,
                                preferred_element_type=jnp.float32)
        t = jnp.maximum(t + d1b_ref[...], 0.0)
        t = jnp.dot(t, d2w_ref[...], preferred_element_type=jnp.float32)
        t = jnp.maximum(t + d2b_ref[...], 0.0)
        o = jnp.dot(t, d3w_ref[...], preferred_element_type=jnp.float32)
        out_ref[...] = jax.nn.sigmoid(o + d3b_ref[...])


@functools.partial(jax.jit, static_argnames=())
def kernel(in_feat, e_weight, W1, b1, W2, b2, W3, b3, D1w, D1b, D2w, D2b,
           D3w, D3b):
    adj = e_weight.reshape(NB, BR, 32, 128)

    full = lambda shape: pl.BlockSpec(shape, lambda p: (0, 0))
    out = pl.pallas_call(
        _gcn_kernel,
        grid=(NB + 1,),
        in_specs=[
            pl.BlockSpec(memory_space=pltpu.MemorySpace.HBM),
            full((N, D)),
            full((D, D)),
            full((D, 1)),
            full((D, D)),
            full((D, 1)),
            full((D, D)),
            full((D, 1)),
            full((D, 16)),
            full((1, 16)),
            full((16, 8)),
            full((1, 8)),
            full((8, 1)),
            full((1, 1)),
        ],
        out_specs=pl.BlockSpec((1, 1), lambda p: (0, 0)),
        out_shape=jax.ShapeDtypeStruct((1, 1), jnp.float32),
        scratch_shapes=[
            pltpu.VMEM((BR, N), jnp.float32),
            pltpu.VMEM((BR, N), jnp.float32),
            pltpu.VMEM((N, N), jnp.bfloat16),
            pltpu.VMEM((D, N), jnp.bfloat16),
            pltpu.VMEM((D, N), jnp.bfloat16),
            pltpu.VMEM((D, N), jnp.float32),
            pltpu.VMEM((1, N), jnp.float32),
            pltpu.SemaphoreType.DMA((2,)),
        ],
    )(adj, in_feat, W1, b1.reshape(D, 1), W2, b2.reshape(D, 1), W3,
      b3.reshape(D, 1), D1w, D1b.reshape(1, 16), D2w, D2b.reshape(1, 8),
      D3w, D3b.reshape(1, 1))
    return out
